# baseline (speedup 1.0000x reference)
import functools

import jax
import jax.numpy as jnp
from jax.experimental import pallas as pl

E = 64
K = 8
DI1 = 64
DI2 = 32
DO1 = 64
DO2 = 32
DIN = DI1 * DI2
DOUT = DO1 * DO2

BLK = 16


def _topk_weights(logits):
    cur = logits
    top1 = jnp.max(cur, axis=-1, keepdims=True)
    wacc = jnp.zeros_like(logits)
    denom = jnp.zeros_like(top1)
    iota = jax.lax.broadcasted_iota(jnp.int32, logits.shape, 1)
    for _ in range(K):
        m = jnp.max(cur, axis=-1, keepdims=True)
        sel = cur == m
        midx = jnp.min(jnp.where(sel, iota, E), axis=-1, keepdims=True)
        first = iota == midx
        ev = jnp.exp(m - top1)
        wacc = wacc + jnp.where(first, ev, 0.0)
        denom = denom + ev
        cur = jnp.where(first, -jnp.inf, cur)
    return wacc / denom


def _moe_kernel(x_ref, wrt_ref, sa_ref, sb_ref, sc_ref, bias_ref, out_ref):
    m = x_ref.shape[0]
    xb = x_ref[...]

    logits = jnp.dot(xb, wrt_ref[...], preferred_element_type=jnp.float32)
    w = _topk_weights(logits)

    xt = xb.reshape(m, DI1, DI2).swapaxes(1, 2).reshape(m * DI2, DI1)
    t = jnp.dot(xt.astype(jnp.bfloat16), sa_ref[...],
                preferred_element_type=jnp.float32)

    wexp = jnp.broadcast_to(w.reshape(m, 1, E, 1), (m, DI2, E, DO1))
    t = t.reshape(m, DI2, E, DO1) * wexp

    t5 = t.transpose(0, 3, 2, 1).reshape(m * DO1, E * DI2).astype(jnp.bfloat16)
    out = jnp.dot(t5, sb_ref[...], preferred_element_type=jnp.float32)

    out_ref[...] = out.reshape(m, DO1, DO2) * sc_ref[0, 0] + bias_ref[...]


@jax.jit
def _run(xf, wrt, sa, sb, scale2, bias2):
    n = xf.shape[0]
    grid = (n // BLK,)
    return pl.pallas_call(
        _moe_kernel,
        grid=grid,
        in_specs=[
            pl.BlockSpec((BLK, DIN), lambda i: (i, 0)),
            pl.BlockSpec((DIN, E), lambda i: (0, 0)),
            pl.BlockSpec((DI1, E * DO1), lambda i: (0, 0)),
            pl.BlockSpec((E * DI2, DO2), lambda i: (0, 0)),
            pl.BlockSpec((1, 1), lambda i: (0, 0)),
            pl.BlockSpec((1, DO1, DO2), lambda i: (0, 0, 0)),
        ],
        out_specs=pl.BlockSpec((BLK, DO1, DO2), lambda i: (i, 0, 0)),
        out_shape=jax.ShapeDtypeStruct((n, DO1, DO2), jnp.float32),
    )(xf, wrt, sa, sb, scale2, bias2)


def kernel(x, Wr, A, B, scale, bias):
    orig_shape = x.shape
    xf = x.reshape(-1, DIN)
    wrt = Wr.T
    sa = A.transpose(2, 0, 1).reshape(DI1, E * DO1).astype(jnp.bfloat16)
    sb = B.transpose(0, 2, 1).reshape(E * DI2, DO2).astype(jnp.bfloat16)
    out = _run(xf, wrt, sa, sb, scale.reshape(1, 1), bias.reshape(1, DO1, DO2))
    out = out.reshape(*orig_shape[:-1], DOUT)
    aux_loss = jnp.asarray(0.0, dtype=x.dtype)
    return (out, aux_loss)


# speedup vs baseline: 1.7650x; 1.7650x over previous
import functools

import jax
import jax.numpy as jnp
from jax.experimental import pallas as pl

E = 64
K = 8
DI1 = 64
DI2 = 32
DO1 = 64
DO2 = 32
DIN = DI1 * DI2
DOUT = DO1 * DO2

BLK = 32


def _topk_weights(logits):
    cur = logits
    top1 = jnp.max(cur, axis=-1, keepdims=True)
    wacc = jnp.zeros_like(logits)
    denom = jnp.zeros_like(top1)
    iota = jax.lax.broadcasted_iota(jnp.int32, logits.shape, 1)
    for _ in range(K):
        m = jnp.max(cur, axis=-1, keepdims=True)
        sel = cur == m
        midx = jnp.min(jnp.where(sel, iota, E), axis=-1, keepdims=True)
        first = iota == midx
        ev = jnp.exp(m - top1)
        wacc = wacc + jnp.where(first, ev, 0.0)
        denom = denom + ev
        cur = jnp.where(first, -jnp.inf, cur)
    return wacc / denom


def _moe_kernel(x_ref, wrt_ref, sa_ref, sb_ref, sc_ref, bias_ref, out_ref):
    m = x_ref.shape[0]
    xb = x_ref[...]

    logits = jnp.dot(xb, wrt_ref[...], preferred_element_type=jnp.float32)
    w = _topk_weights(logits)

    xt = xb.reshape(m, DI1, DI2).swapaxes(1, 2).reshape(m * DI2, DI1)
    t = jnp.dot(xt.astype(jnp.bfloat16), sa_ref[...],
                preferred_element_type=jnp.float32).astype(jnp.bfloat16)

    wexp = jnp.broadcast_to(w.astype(jnp.bfloat16).reshape(m, 1, E, 1),
                            (m, DI2, E, DO1))
    t = t.reshape(m, DI2, E, DO1) * wexp

    t5 = t.transpose(0, 3, 2, 1).reshape(m * DO1, E * DI2)
    out = jnp.dot(t5, sb_ref[...], preferred_element_type=jnp.float32)

    out_ref[...] = out.reshape(m, DO1, DO2) * sc_ref[0, 0] + bias_ref[...]


@jax.jit
def _run(xf, wrt, sa, sb, scale2, bias2):
    n = xf.shape[0]
    grid = (n // BLK,)
    return pl.pallas_call(
        _moe_kernel,
        grid=grid,
        in_specs=[
            pl.BlockSpec((BLK, DIN), lambda i: (i, 0)),
            pl.BlockSpec((DIN, E), lambda i: (0, 0)),
            pl.BlockSpec((DI1, E * DO1), lambda i: (0, 0)),
            pl.BlockSpec((E * DI2, DO2), lambda i: (0, 0)),
            pl.BlockSpec((1, 1), lambda i: (0, 0)),
            pl.BlockSpec((1, DO1, DO2), lambda i: (0, 0, 0)),
        ],
        out_specs=pl.BlockSpec((BLK, DO1, DO2), lambda i: (i, 0, 0)),
        out_shape=jax.ShapeDtypeStruct((n, DO1, DO2), jnp.float32),
    )(xf, wrt, sa, sb, scale2, bias2)


def kernel(x, Wr, A, B, scale, bias):
    orig_shape = x.shape
    xf = x.reshape(-1, DIN)
    wrt = Wr.T
    sa = A.transpose(2, 0, 1).reshape(DI1, E * DO1).astype(jnp.bfloat16)
    sb = B.transpose(0, 2, 1).reshape(E * DI2, DO2).astype(jnp.bfloat16)
    out = _run(xf, wrt, sa, sb, scale.reshape(1, 1), bias.reshape(1, DO1, DO2))
    out = out.reshape(*orig_shape[:-1], DOUT)
    aux_loss = jnp.asarray(0.0, dtype=x.dtype)
    return (out, aux_loss)


# speedup vs baseline: 2.7164x; 1.5391x over previous
import functools

import jax
import jax.numpy as jnp
from jax.experimental import pallas as pl

E = 64
K = 8
DI1 = 64
DI2 = 32
DO1 = 64
DO2 = 32
DIN = DI1 * DI2
DOUT = DO1 * DO2

BLK = 32


def _topk_weights(logits):
    cur = logits
    top1 = jnp.max(cur, axis=-1, keepdims=True)
    wacc = jnp.zeros_like(logits)
    denom = jnp.zeros_like(top1)
    iota = jax.lax.broadcasted_iota(jnp.int32, logits.shape, 1)
    for _ in range(K):
        m = jnp.max(cur, axis=-1, keepdims=True)
        sel = cur == m
        midx = jnp.min(jnp.where(sel, iota, E), axis=-1, keepdims=True)
        first = iota == midx
        ev = jnp.exp(m - top1)
        wacc = wacc + jnp.where(first, ev, 0.0)
        denom = denom + ev
        cur = jnp.where(first, -jnp.inf, cur)
    return wacc / denom


def _moe_kernel(x_ref, wrt_ref, sa_ref, sb_ref, sc_ref, bias_ref, out_ref):
    m = x_ref.shape[0]
    xb = x_ref[...]

    logits = jnp.dot(xb, wrt_ref[...], preferred_element_type=jnp.float32)
    w = _topk_weights(logits)

    xt = xb.reshape(m, DI1, DI2).swapaxes(1, 2).reshape(m * DI2, DI1)
    t = jnp.dot(xt.astype(jnp.bfloat16), sa_ref[...],
                preferred_element_type=jnp.float32).astype(jnp.bfloat16)

    wexp = jnp.broadcast_to(w.astype(jnp.bfloat16).reshape(m, 1, 1, E),
                            (m, DI2, DO1, E))
    t = t.reshape(m, DI2, DO1, E) * wexp

    t5 = t.transpose(0, 2, 1, 3).reshape(m * DO1, DI2 * E)
    out = jnp.dot(t5, sb_ref[...], preferred_element_type=jnp.float32)

    out_ref[...] = out.reshape(m, DO1, DO2) * sc_ref[0, 0] + bias_ref[...]


@jax.jit
def _run(xf, wrt, sa, sb, scale2, bias2):
    n = xf.shape[0]
    grid = (n // BLK,)
    return pl.pallas_call(
        _moe_kernel,
        grid=grid,
        in_specs=[
            pl.BlockSpec((BLK, DIN), lambda i: (i, 0)),
            pl.BlockSpec((DIN, E), lambda i: (0, 0)),
            pl.BlockSpec((DI1, E * DO1), lambda i: (0, 0)),
            pl.BlockSpec((E * DI2, DO2), lambda i: (0, 0)),
            pl.BlockSpec((1, 1), lambda i: (0, 0)),
            pl.BlockSpec((1, DO1, DO2), lambda i: (0, 0, 0)),
        ],
        out_specs=pl.BlockSpec((BLK, DO1, DO2), lambda i: (i, 0, 0)),
        out_shape=jax.ShapeDtypeStruct((n, DO1, DO2), jnp.float32),
    )(xf, wrt, sa, sb, scale2, bias2)


def kernel(x, Wr, A, B, scale, bias):
    orig_shape = x.shape
    xf = x.reshape(-1, DIN)
    wrt = Wr.T
    sa = A.transpose(2, 1, 0).reshape(DI1, DO1 * E).astype(jnp.bfloat16)
    sb = B.transpose(2, 0, 1).reshape(DI2 * E, DO2).astype(jnp.bfloat16)
    out = _run(xf, wrt, sa, sb, scale.reshape(1, 1), bias.reshape(1, DO1, DO2))
    out = out.reshape(*orig_shape[:-1], DOUT)
    aux_loss = jnp.asarray(0.0, dtype=x.dtype)
    return (out, aux_loss)
